# Initial kernel scaffold; baseline (speedup 1.0000x reference)
import jax
import jax.numpy as jnp
from jax.experimental import pallas as pl


def kernel(probas, labels):
    raise NotImplementedError("write your pallas kernel here")


# speedup vs baseline: 9.5450x; 9.5450x over previous
import functools

import jax
import jax.numpy as jnp
from jax.experimental import pallas as pl
from jax.experimental.pallas import tpu as pltpu
from jax.experimental.pallas import tpu_sc as plsc

_B, _C, _D, _H, _W = 2, 5, 32, 256, 256
_NVOX = _D * _H * _W
_N = _B * _NVOX

_SHIFT = 21
_K = 1 << (32 - _SHIFT)
_NW = 32
_LANES = 16
_TBL = 3 * _LANES * _K
_PER_W = _N // _NW
_CHUNK = 4096
_NPAIR = _PER_W // (2 * _CHUNK)

_BV = 131072


def _elemwise_body(prob_ref, lab_ref, out_ref):
    p = jnp.exp(prob_ref[...])
    lab = lab_ref[...]
    cls = jax.lax.broadcasted_iota(jnp.int32, (_B, _C, _BV), 1)
    fg = lab[:, None, :] == cls
    e = jnp.abs(fg.astype(jnp.float32) - p)
    bits = jax.lax.bitcast_convert_type(e, jnp.int32)
    bits = jnp.where(fg, bits | 1, bits & -2)
    out_ref[...] = jax.lax.bitcast_convert_type(bits, jnp.float32)


def _elemwise(probas, labels):
    grid = _NVOX // _BV
    return pl.pallas_call(
        _elemwise_body,
        grid=(grid,),
        in_specs=[
            pl.BlockSpec((_B, _C, _BV), lambda i: (0, 0, i)),
            pl.BlockSpec((_B, _BV), lambda i: (0, i)),
        ],
        out_specs=pl.BlockSpec((_B, _C, _BV), lambda i: (0, 0, i)),
        out_shape=jax.ShapeDtypeStruct((_B, _C, _NVOX), jnp.float32),
    )(probas, labels)


def _histo_body(e_hbm, out_hbm, tbl, buf0, buf1, sem0, sem1):
    cid = jax.lax.axis_index("c")
    sid = jax.lax.axis_index("s")
    wid = sid * 2 + cid
    half = wid // 16
    sub = wid % 16
    lane = jax.lax.iota(jnp.int32, 16)
    ones = jnp.ones((16,), jnp.float32)

    def process(buf):
        def body(i, carry):
            v = buf[pl.ds(i * 16, 16)]
            bits = plsc.bitcast(v, jnp.int32)
            key = jax.lax.shift_right_logical(bits, _SHIFT)
            fgf = (bits & 1).astype(jnp.float32)
            idx = lane * _K + key
            plsc.addupdate_scatter(tbl, [idx], ones)
            plsc.addupdate_scatter(tbl, [idx + _LANES * _K], fgf)
            plsc.addupdate_scatter(tbl, [idx + 2 * _LANES * _K], v)
            return carry
        jax.lax.fori_loop(0, _CHUNK // 16, body, 0)

    for c in range(_C):
        def zero(i, carry):
            tbl[pl.ds(i * 16, 16)] = jnp.zeros((16,), jnp.float32)
            return carry
        jax.lax.fori_loop(0, _TBL // 16, zero, 0)

        start = half * (_C * _NVOX) + c * _NVOX + sub * _PER_W
        pltpu.async_copy(e_hbm.at[pl.ds(start, _CHUNK)], buf0, sem0)
        pltpu.async_copy(e_hbm.at[pl.ds(start + _CHUNK, _CHUNK)], buf1, sem1)

        def pair(jp, carry):
            base = start + jp * 2 * _CHUNK
            pltpu.make_async_copy(
                e_hbm.at[pl.ds(0, _CHUNK)], buf0, sem0).wait()
            process(buf0)

            @pl.when(jp < _NPAIR - 1)
            def _():
                pltpu.async_copy(
                    e_hbm.at[pl.ds(base + 2 * _CHUNK, _CHUNK)], buf0, sem0)

            pltpu.make_async_copy(
                e_hbm.at[pl.ds(0, _CHUNK)], buf1, sem1).wait()
            process(buf1)

            @pl.when(jp < _NPAIR - 1)
            def _():
                pltpu.async_copy(
                    e_hbm.at[pl.ds(base + 3 * _CHUNK, _CHUNK)], buf1, sem1)

            return carry
        jax.lax.fori_loop(0, _NPAIR, pair, 0)

        pltpu.sync_copy(tbl, out_hbm.at[c, wid])


def _histogram(e_flat):
    mesh = plsc.VectorSubcoreMesh(core_axis_name="c", subcore_axis_name="s")
    kern = functools.partial(
        pl.kernel,
        out_type=jax.ShapeDtypeStruct((_C, _NW, _TBL), jnp.float32),
        mesh=mesh,
        compiler_params=pltpu.CompilerParams(needs_layout_passes=False),
        scratch_types=[
            pltpu.VMEM((_TBL,), jnp.float32),
            pltpu.VMEM((_CHUNK,), jnp.float32),
            pltpu.VMEM((_CHUNK,), jnp.float32),
            pltpu.SemaphoreType.DMA,
            pltpu.SemaphoreType.DMA,
        ],
    )(_histo_body)
    return kern(e_flat)


def _shift_cols(x, s):
    return jnp.concatenate(
        [jnp.zeros((x.shape[0], s), x.dtype), x[:, : x.shape[1] - s]], axis=1)


def _shift_rows(x, s):
    return jnp.concatenate(
        [jnp.zeros((s, x.shape[1]), x.dtype), x[: x.shape[0] - s]], axis=0)


def _cumsum2d(x):
    for s in (1, 2, 4, 8, 16, 32, 64):
        x = x + _shift_cols(x, s)
    rows = x[:, 127:128]
    r = rows
    for s in (1, 2, 4, 8):
        r = r + _shift_rows(r, s)
    return x + (r - rows)


def _final_body(h_ref, out_ref, acc_ref):
    c = pl.program_id(0)
    xs = jnp.sum(h_ref[0], axis=0)
    xs2 = xs.reshape(_TBL // 128, 128)

    def tab(t):
        y = xs2[t * 256:(t + 1) * 256].reshape(16, 16, 128)
        return jnp.sum(y, axis=0)

    m, f, s_sum = tab(0), tab(1), tab(2)
    incl_m = _cumsum2d(m)
    incl_f = _cumsum2d(f)
    g_tot = incl_f[15, 127]
    k0 = float(_N) - incl_m
    f0 = g_tot - incl_f
    u0 = g_tot + k0 - f0
    u1 = g_tot + (k0 + m) - (f0 + f)
    num = m * (g_tot - f0) + k0 * f
    den = u0 * u1
    d_j = jnp.where(den > 0, num / jnp.maximum(den, 1.0), 0.0)
    contrib = jnp.where(m > 0, s_sum * d_j / jnp.maximum(m, 1.0), 0.0)
    loss_c = jnp.sum(contrib)
    pres = (g_tot > 0).astype(jnp.float32)

    @pl.when(c == 0)
    def _():
        acc_ref[0] = 0.0
        acc_ref[1] = 0.0

    acc_ref[0] += loss_c * pres
    acc_ref[1] += pres

    @pl.when(c == _C - 1)
    def _():
        out_ref[...] = jnp.full((1, 1), acc_ref[0] / acc_ref[1], jnp.float32)


def _finalize(hist):
    return pl.pallas_call(
        _final_body,
        grid=(_C,),
        in_specs=[pl.BlockSpec((1, _NW, _TBL), lambda c: (c, 0, 0))],
        out_specs=pl.BlockSpec((1, 1), lambda c: (0, 0)),
        out_shape=jax.ShapeDtypeStruct((1, 1), jnp.float32),
        scratch_shapes=[pltpu.SMEM((2,), jnp.float32)],
    )(hist)


def kernel(probas, labels):
    probas3 = probas.reshape(_B, _C, _NVOX)
    labels2 = labels.reshape(_B, _NVOX)
    e_packed = _elemwise(probas3, labels2)
    hist = _histogram(e_packed.reshape(-1))
    out = _finalize(hist)
    return out.reshape(())


# speedup vs baseline: 101.4316x; 10.6267x over previous
import functools

import jax
import jax.numpy as jnp
from jax.experimental import pallas as pl
from jax.experimental.pallas import tpu as pltpu
from jax.experimental.pallas import tpu_sc as plsc

_B, _C, _D, _H, _W = 2, 5, 32, 256, 256
_NVOX = _D * _H * _W
_N = _B * _NVOX

_SHIFT = 21
_K = 1 << (32 - _SHIFT)
_NW = 32
_LANES = 16
_TBL = 3 * _LANES * _K
_RED = 3 * _K
_PER_W = _N // _NW
_CHUNK = 4096
_NPAIR = _PER_W // (2 * _CHUNK)

_BV = 131072


def _histo_body(p_hbm, l_hbm, out_hbm,
                tbl_cf, tbl_e, red,
                bufp0, bufp1, bufl0, bufl1,
                semp0, semp1, seml0, seml1):
    cid = jax.lax.axis_index("c")
    sid = jax.lax.axis_index("s")
    wid = sid * 2 + cid
    half = wid // 16
    sub = wid % 16
    lstart = half * _NVOX + sub * _PER_W
    lane_off = jax.lax.iota(jnp.int32, 16) * _K
    one_f = jnp.full((16,), 1.0, jnp.float32)
    zero_f = jnp.zeros((16,), jnp.float32)
    pk_bg = jnp.full((16,), 1, jnp.int32)
    pk_fg = jnp.full((16,), 1 + (1 << 17), jnp.int32)

    def process(bufp, bufl, c):
        def body(i, carry):
            pv = [bufp[pl.ds(i * 128 + u * 16, 16)] for u in range(8)]
            lv = [bufl[pl.ds(i * 128 + u * 16, 16)] for u in range(8)]
            mks = [l == c for l in lv]
            es = [jnp.abs(jnp.where(m, one_f, zero_f) - jnp.exp(p))
                  for m, p in zip(mks, pv)]
            idxs = [lane_off + jax.lax.shift_right_logical(
                plsc.bitcast(e, jnp.int32), _SHIFT) for e in es]
            for e, idx, m in zip(es, idxs, mks):
                plsc.addupdate_scatter(
                    tbl_cf, [idx], jnp.where(m, pk_fg, pk_bg))
                plsc.addupdate_scatter(tbl_e, [idx], e)
            return carry
        jax.lax.fori_loop(0, _CHUNK // 128, body, 0)

    for c in range(_C):
        def zero(i, carry):
            tbl_cf[pl.ds(i * 16, 16)] = jnp.zeros((16,), jnp.int32)
            tbl_e[pl.ds(i * 16, 16)] = jnp.zeros((16,), jnp.float32)
            return carry
        jax.lax.fori_loop(0, _LANES * _K // 16, zero, 0)

        start = half * (_C * _NVOX) + c * _NVOX + sub * _PER_W
        pltpu.async_copy(p_hbm.at[pl.ds(start, _CHUNK)], bufp0, semp0)
        pltpu.async_copy(l_hbm.at[pl.ds(lstart, _CHUNK)], bufl0, seml0)
        pltpu.async_copy(p_hbm.at[pl.ds(start + _CHUNK, _CHUNK)], bufp1, semp1)
        pltpu.async_copy(l_hbm.at[pl.ds(lstart + _CHUNK, _CHUNK)], bufl1, seml1)

        def pair(jp, carry):
            base = start + jp * 2 * _CHUNK
            lbase = lstart + jp * 2 * _CHUNK
            pltpu.make_async_copy(
                p_hbm.at[pl.ds(0, _CHUNK)], bufp0, semp0).wait()
            pltpu.make_async_copy(
                l_hbm.at[pl.ds(0, _CHUNK)], bufl0, seml0).wait()
            process(bufp0, bufl0, c)

            @pl.when(jp < _NPAIR - 1)
            def _():
                pltpu.async_copy(
                    p_hbm.at[pl.ds(base + 2 * _CHUNK, _CHUNK)], bufp0, semp0)
                pltpu.async_copy(
                    l_hbm.at[pl.ds(lbase + 2 * _CHUNK, _CHUNK)], bufl0, seml0)

            pltpu.make_async_copy(
                p_hbm.at[pl.ds(0, _CHUNK)], bufp1, semp1).wait()
            pltpu.make_async_copy(
                l_hbm.at[pl.ds(0, _CHUNK)], bufl1, seml1).wait()
            process(bufp1, bufl1, c)

            @pl.when(jp < _NPAIR - 1)
            def _():
                pltpu.async_copy(
                    p_hbm.at[pl.ds(base + 3 * _CHUNK, _CHUNK)], bufp1, semp1)
                pltpu.async_copy(
                    l_hbm.at[pl.ds(lbase + 3 * _CHUNK, _CHUNK)], bufl1, seml1)

            return carry
        jax.lax.fori_loop(0, _NPAIR, pair, 0)

        def lred_cf(i, carry):
            acc_m = jnp.zeros((16,), jnp.int32)
            acc_f = jnp.zeros((16,), jnp.int32)
            for l in range(_LANES):
                v = tbl_cf[pl.ds(l * _K + i * 16, 16)]
                acc_m = acc_m + (v & ((1 << 17) - 1))
                acc_f = acc_f + jax.lax.shift_right_logical(v, 17)
            red[pl.ds(i * 16, 16)] = acc_m.astype(jnp.float32)
            red[pl.ds(_K + i * 16, 16)] = acc_f.astype(jnp.float32)
            return carry
        jax.lax.fori_loop(0, _K // 16, lred_cf, 0)

        def lred_e(i, carry):
            acc = tbl_e[pl.ds(i * 16, 16)]
            for l in range(1, _LANES):
                acc = acc + tbl_e[pl.ds(l * _K + i * 16, 16)]
            red[pl.ds(2 * _K + i * 16, 16)] = acc
            return carry
        jax.lax.fori_loop(0, _K // 16, lred_e, 0)

        pltpu.sync_copy(red, out_hbm.at[pl.ds((c * _NW + wid) * _RED, _RED)])


def _histogram(probas_flat, labels_flat):
    mesh = plsc.VectorSubcoreMesh(core_axis_name="c", subcore_axis_name="s")
    kern = functools.partial(
        pl.kernel,
        out_type=jax.ShapeDtypeStruct((_C * _NW * _RED,), jnp.float32),
        mesh=mesh,
        compiler_params=pltpu.CompilerParams(
            needs_layout_passes=False, disable_bounds_checks=True),
        scratch_types=[
            pltpu.VMEM((_LANES * _K,), jnp.int32),
            pltpu.VMEM((_LANES * _K,), jnp.float32),
            pltpu.VMEM((_RED,), jnp.float32),
            pltpu.VMEM((_CHUNK,), jnp.float32),
            pltpu.VMEM((_CHUNK,), jnp.float32),
            pltpu.VMEM((_CHUNK,), jnp.int32),
            pltpu.VMEM((_CHUNK,), jnp.int32),
            pltpu.SemaphoreType.DMA,
            pltpu.SemaphoreType.DMA,
            pltpu.SemaphoreType.DMA,
            pltpu.SemaphoreType.DMA,
        ],
    )(_histo_body)
    return kern(probas_flat, labels_flat)


def _shift_cols(x, s):
    return jnp.concatenate(
        [jnp.zeros((x.shape[0], s), x.dtype), x[:, : x.shape[1] - s]], axis=1)


def _shift_rows(x, s):
    return jnp.concatenate(
        [jnp.zeros((s, x.shape[1]), x.dtype), x[: x.shape[0] - s]], axis=0)


def _cumsum2d(x):
    for s in (1, 2, 4, 8, 16, 32, 64):
        x = x + _shift_cols(x, s)
    rows = x[:, 127:128]
    r = rows
    for s in (1, 2, 4, 8):
        r = r + _shift_rows(r, s)
    return x + (r - rows)


def _final_body(h_ref, out_ref, acc_ref):
    c = pl.program_id(0)
    xs = h_ref[...].reshape(_NW, _RED // 128, 128)
    m3 = jnp.sum(xs, axis=0)
    m = m3[0:16]
    f = m3[16:32]
    s_sum = m3[32:48]
    incl_m = _cumsum2d(m)
    incl_f = _cumsum2d(f)
    g_tot = incl_f[15, 127]
    k0 = float(_N) - incl_m
    f0 = g_tot - incl_f
    u0 = g_tot + k0 - f0
    u1 = g_tot + (k0 + m) - (f0 + f)
    num = m * (g_tot - f0) + k0 * f
    den = u0 * u1
    d_j = jnp.where(den > 0, num / jnp.maximum(den, 1.0), 0.0)
    contrib = jnp.where(m > 0, s_sum * d_j / jnp.maximum(m, 1.0), 0.0)
    loss_c = jnp.sum(contrib)
    pres = (g_tot > 0).astype(jnp.float32)

    @pl.when(c == 0)
    def _():
        acc_ref[0] = 0.0
        acc_ref[1] = 0.0

    acc_ref[0] += loss_c * pres
    acc_ref[1] += pres

    @pl.when(c == _C - 1)
    def _():
        out_ref[...] = jnp.full((1, 1), acc_ref[0] / acc_ref[1], jnp.float32)


def _finalize(hist):
    return pl.pallas_call(
        _final_body,
        grid=(_C,),
        in_specs=[pl.BlockSpec((_NW * _RED,), lambda c: (c,))],
        out_specs=pl.BlockSpec((1, 1), lambda c: (0, 0)),
        out_shape=jax.ShapeDtypeStruct((1, 1), jnp.float32),
        scratch_shapes=[pltpu.SMEM((2,), jnp.float32)],
    )(hist)


def kernel(probas, labels):
    probas_flat = probas.reshape(-1)
    labels_flat = labels.reshape(-1)
    hist = _histogram(probas_flat, labels_flat)
    out = _finalize(hist)
    return out.reshape(())


# speedup vs baseline: 103.2523x; 1.0180x over previous
import functools

import jax
import jax.numpy as jnp
from jax.experimental import pallas as pl
from jax.experimental.pallas import tpu as pltpu
from jax.experimental.pallas import tpu_sc as plsc

_B, _C, _D, _H, _W = 2, 5, 32, 256, 256
_NVOX = _D * _H * _W
_N = _B * _NVOX

_SHIFT = 21
_K = 1 << (32 - _SHIFT)
_NW = 32
_LANES = 16
_TBL = 3 * _LANES * _K
_RED = 3 * _K
_PER_W = _N // _NW
_CHUNK = 4096
_NPAIR = _PER_W // (2 * _CHUNK)

_BV = 131072


def _histo_body(p_hbm, l_hbm, out_hbm,
                tbl_cf, tbl_e, red,
                bufp0, bufp1, bufl0, bufl1,
                semp0, semp1, seml0, seml1):
    cid = jax.lax.axis_index("c")
    sid = jax.lax.axis_index("s")
    wid = sid * 2 + cid
    half = wid // 16
    sub = wid % 16
    lstart = half * _NVOX + sub * _PER_W
    lane_iota = jax.lax.iota(jnp.int32, 16)
    lane_off = lane_iota * _K
    one_f = jnp.full((16,), 1.0, jnp.float32)
    zero_f = jnp.zeros((16,), jnp.float32)
    pk_bg = jnp.full((16,), 1, jnp.int32)
    pk_fg = jnp.full((16,), 1 + (1 << 17), jnp.int32)

    def process(bufp, bufl, c):
        def body(i, carry):
            pv = [bufp[pl.ds(i * 128 + u * 16, 16)] for u in range(8)]
            lv = [bufl[pl.ds(i * 128 + u * 16, 16)] for u in range(8)]
            mks = [l == c for l in lv]
            es = [jnp.abs(jnp.where(m, one_f, zero_f) - jnp.exp(p))
                  for m, p in zip(mks, pv)]
            idxs = [lane_off + (lane_iota ^ jax.lax.shift_right_logical(
                plsc.bitcast(e, jnp.int32), _SHIFT)) for e in es]
            for e, idx, m in zip(es, idxs, mks):
                plsc.addupdate_scatter(
                    tbl_cf, [idx], jnp.where(m, pk_fg, pk_bg))
                plsc.addupdate_scatter(tbl_e, [idx], e)
            return carry
        jax.lax.fori_loop(0, _CHUNK // 128, body, 0)

    for c in range(_C):
        def zero(i, carry):
            tbl_cf[pl.ds(i * 16, 16)] = jnp.zeros((16,), jnp.int32)
            tbl_e[pl.ds(i * 16, 16)] = jnp.zeros((16,), jnp.float32)
            return carry
        jax.lax.fori_loop(0, _LANES * _K // 16, zero, 0)

        start = half * (_C * _NVOX) + c * _NVOX + sub * _PER_W
        pltpu.async_copy(p_hbm.at[pl.ds(start, _CHUNK)], bufp0, semp0)
        pltpu.async_copy(l_hbm.at[pl.ds(lstart, _CHUNK)], bufl0, seml0)
        pltpu.async_copy(p_hbm.at[pl.ds(start + _CHUNK, _CHUNK)], bufp1, semp1)
        pltpu.async_copy(l_hbm.at[pl.ds(lstart + _CHUNK, _CHUNK)], bufl1, seml1)

        def pair(jp, carry):
            base = start + jp * 2 * _CHUNK
            lbase = lstart + jp * 2 * _CHUNK
            pltpu.make_async_copy(
                p_hbm.at[pl.ds(0, _CHUNK)], bufp0, semp0).wait()
            pltpu.make_async_copy(
                l_hbm.at[pl.ds(0, _CHUNK)], bufl0, seml0).wait()
            process(bufp0, bufl0, c)

            @pl.when(jp < _NPAIR - 1)
            def _():
                pltpu.async_copy(
                    p_hbm.at[pl.ds(base + 2 * _CHUNK, _CHUNK)], bufp0, semp0)
                pltpu.async_copy(
                    l_hbm.at[pl.ds(lbase + 2 * _CHUNK, _CHUNK)], bufl0, seml0)

            pltpu.make_async_copy(
                p_hbm.at[pl.ds(0, _CHUNK)], bufp1, semp1).wait()
            pltpu.make_async_copy(
                l_hbm.at[pl.ds(0, _CHUNK)], bufl1, seml1).wait()
            process(bufp1, bufl1, c)

            @pl.when(jp < _NPAIR - 1)
            def _():
                pltpu.async_copy(
                    p_hbm.at[pl.ds(base + 3 * _CHUNK, _CHUNK)], bufp1, semp1)
                pltpu.async_copy(
                    l_hbm.at[pl.ds(lbase + 3 * _CHUNK, _CHUNK)], bufl1, seml1)

            return carry
        jax.lax.fori_loop(0, _NPAIR, pair, 0)

        xor_iotas = [jnp.arange(16, dtype=jnp.int32) ^ l
                     for l in range(_LANES)]

        def lred_cf(i, carry):
            acc_m = jnp.zeros((16,), jnp.int32)
            acc_f = jnp.zeros((16,), jnp.int32)
            for l in range(_LANES):
                v = plsc.load_gather(
                    tbl_cf, [l * _K + i * 16 + xor_iotas[l]])
                acc_m = acc_m + (v & ((1 << 17) - 1))
                acc_f = acc_f + jax.lax.shift_right_logical(v, 17)
            red[pl.ds(i * 16, 16)] = acc_m.astype(jnp.float32)
            red[pl.ds(_K + i * 16, 16)] = acc_f.astype(jnp.float32)
            return carry
        jax.lax.fori_loop(0, _K // 16, lred_cf, 0)

        def lred_e(i, carry):
            acc = jnp.zeros((16,), jnp.float32)
            for l in range(_LANES):
                acc = acc + plsc.load_gather(
                    tbl_e, [l * _K + i * 16 + xor_iotas[l]])
            red[pl.ds(2 * _K + i * 16, 16)] = acc
            return carry
        jax.lax.fori_loop(0, _K // 16, lred_e, 0)

        pltpu.sync_copy(red, out_hbm.at[pl.ds((c * _NW + wid) * _RED, _RED)])


def _histogram(probas_flat, labels_flat):
    mesh = plsc.VectorSubcoreMesh(core_axis_name="c", subcore_axis_name="s")
    kern = functools.partial(
        pl.kernel,
        out_type=jax.ShapeDtypeStruct((_C * _NW * _RED,), jnp.float32),
        mesh=mesh,
        compiler_params=pltpu.CompilerParams(
            needs_layout_passes=False, disable_bounds_checks=True),
        scratch_types=[
            pltpu.VMEM((_LANES * _K,), jnp.int32),
            pltpu.VMEM((_LANES * _K,), jnp.float32),
            pltpu.VMEM((_RED,), jnp.float32),
            pltpu.VMEM((_CHUNK,), jnp.float32),
            pltpu.VMEM((_CHUNK,), jnp.float32),
            pltpu.VMEM((_CHUNK,), jnp.int32),
            pltpu.VMEM((_CHUNK,), jnp.int32),
            pltpu.SemaphoreType.DMA,
            pltpu.SemaphoreType.DMA,
            pltpu.SemaphoreType.DMA,
            pltpu.SemaphoreType.DMA,
        ],
    )(_histo_body)
    return kern(probas_flat, labels_flat)


def _shift_cols(x, s):
    return jnp.concatenate(
        [jnp.zeros((x.shape[0], s), x.dtype), x[:, : x.shape[1] - s]], axis=1)


def _shift_rows(x, s):
    return jnp.concatenate(
        [jnp.zeros((s, x.shape[1]), x.dtype), x[: x.shape[0] - s]], axis=0)


def _cumsum2d(x):
    for s in (1, 2, 4, 8, 16, 32, 64):
        x = x + _shift_cols(x, s)
    rows = x[:, 127:128]
    r = rows
    for s in (1, 2, 4, 8):
        r = r + _shift_rows(r, s)
    return x + (r - rows)


def _final_body(h_ref, out_ref, acc_ref):
    c = pl.program_id(0)
    xs = h_ref[...].reshape(_NW, _RED // 128, 128)
    m3 = jnp.sum(xs, axis=0)
    m = m3[0:16]
    f = m3[16:32]
    s_sum = m3[32:48]
    incl_m = _cumsum2d(m)
    incl_f = _cumsum2d(f)
    g_tot = incl_f[15, 127]
    k0 = float(_N) - incl_m
    f0 = g_tot - incl_f
    u0 = g_tot + k0 - f0
    u1 = g_tot + (k0 + m) - (f0 + f)
    num = m * (g_tot - f0) + k0 * f
    den = u0 * u1
    d_j = jnp.where(den > 0, num / jnp.maximum(den, 1.0), 0.0)
    contrib = jnp.where(m > 0, s_sum * d_j / jnp.maximum(m, 1.0), 0.0)
    loss_c = jnp.sum(contrib)
    pres = (g_tot > 0).astype(jnp.float32)

    @pl.when(c == 0)
    def _():
        acc_ref[0] = 0.0
        acc_ref[1] = 0.0

    acc_ref[0] += loss_c * pres
    acc_ref[1] += pres

    @pl.when(c == _C - 1)
    def _():
        out_ref[...] = jnp.full((1, 1), acc_ref[0] / acc_ref[1], jnp.float32)


def _finalize(hist):
    return pl.pallas_call(
        _final_body,
        grid=(_C,),
        in_specs=[pl.BlockSpec((_NW * _RED,), lambda c: (c,))],
        out_specs=pl.BlockSpec((1, 1), lambda c: (0, 0)),
        out_shape=jax.ShapeDtypeStruct((1, 1), jnp.float32),
        scratch_shapes=[pltpu.SMEM((2,), jnp.float32)],
    )(hist)


def kernel(probas, labels):
    probas_flat = probas.reshape(-1)
    labels_flat = labels.reshape(-1)
    hist = _histogram(probas_flat, labels_flat)
    out = _finalize(hist)
    return out.reshape(())
